# baseline (speedup 1.0000x reference)
import jax
import jax.numpy as jnp
from jax.experimental import pallas as pl
from jax.experimental.pallas import tpu as pltpu

_BLK = 256


def _prep_body(x_ref, wnb_ref, bnb_ref, watt_ref, m_ref):
    B, N, F = x_ref.shape
    w2 = watt_ref[1:2, :]
    es = []
    for b in range(B):
        x = x_ref[b]
        nb = (jnp.dot(x, wnb_ref[...], preferred_element_type=jnp.float32)
              + bnb_ref[...])
        att = jnp.sum(x * w2, axis=1, keepdims=True)
        e = jnp.exp(att - jnp.max(att))
        m_ref[:, b * F:(b + 1) * F] = e * nb
        es.append(e)
    es.append(jnp.zeros((N, F - B), dtype=jnp.float32))
    m_ref[:, B * F:] = jnp.concatenate(es, axis=1)


def _main_body(x_ref, a_ref, m_ref, wself_ref, bself_ref, gamma_ref, beta_ref,
               out_ref):
    B = x_ref.shape[0]
    F = x_ref.shape[2]
    mm = jnp.dot(a_ref[...], m_ref[...], preferred_element_type=jnp.float32)
    for b in range(B):
        self_blk = (jnp.dot(x_ref[b], wself_ref[...],
                            preferred_element_type=jnp.float32)
                    + bself_ref[...])
        num = mm[:, b * F:(b + 1) * F]
        den = mm[:, B * F + b:B * F + b + 1]
        rec = jnp.where(den > 0, 1.0 / den, 0.0)
        comb = self_blk + num * rec
        mean = jnp.mean(comb, axis=1, keepdims=True)
        cent = comb - mean
        var = jnp.mean(cent * cent, axis=1, keepdims=True)
        rstd = jax.lax.rsqrt(var + 1e-5)
        out_ref[b] = jnp.maximum(
            (cent * rstd) * gamma_ref[...] + beta_ref[...], 0.0)


def kernel(node_features, adjacency_matrix, W_self, b_self, W_nb, b_nb,
           W_att, b_att, ln_gamma, ln_beta):
    B, N, F = node_features.shape
    watt2 = W_att.reshape(2, F)
    bself = b_self.reshape(1, F)
    bnb = b_nb.reshape(1, F)
    gamma = ln_gamma.reshape(1, F)
    beta = ln_beta.reshape(1, F)
    MCOLS = (B + 1) * F

    m = pl.pallas_call(
        _prep_body,
        in_specs=[
            pl.BlockSpec((B, N, F), lambda: (0, 0, 0)),
            pl.BlockSpec((F, F), lambda: (0, 0)),
            pl.BlockSpec((1, F), lambda: (0, 0)),
            pl.BlockSpec((2, F), lambda: (0, 0)),
        ],
        out_specs=pl.BlockSpec((N, MCOLS), lambda: (0, 0)),
        out_shape=jax.ShapeDtypeStruct((N, MCOLS), jnp.float32),
    )(node_features, W_nb, bnb, watt2)

    grid = (N // _BLK,)
    out = pl.pallas_call(
        _main_body,
        grid=grid,
        in_specs=[
            pl.BlockSpec((B, _BLK, F), lambda i: (0, i, 0)),
            pl.BlockSpec((_BLK, N), lambda i: (i, 0)),
            pl.BlockSpec((N, MCOLS), lambda i: (0, 0)),
            pl.BlockSpec((F, F), lambda i: (0, 0)),
            pl.BlockSpec((1, F), lambda i: (0, 0)),
            pl.BlockSpec((1, F), lambda i: (0, 0)),
            pl.BlockSpec((1, F), lambda i: (0, 0)),
        ],
        out_specs=pl.BlockSpec((B, _BLK, F), lambda i: (0, i, 0)),
        out_shape=jax.ShapeDtypeStruct((B, N, F), jnp.float32),
        compiler_params=pltpu.CompilerParams(
            dimension_semantics=("arbitrary",),
        ),
    )(node_features, adjacency_matrix, m, W_self, bself, gamma, beta)
    return out


# speedup vs baseline: 1.0020x; 1.0020x over previous
import jax
import jax.numpy as jnp
from jax.experimental import pallas as pl
from jax.experimental.pallas import tpu as pltpu

_BLK = 256


def _prep_body(x_ref, wnb_ref, bnb_ref, watt_ref, m_ref):
    B, N, F = x_ref.shape
    w2 = watt_ref[1:2, :]
    es = []
    for b in range(B):
        x = x_ref[b]
        nb = (jnp.dot(x, wnb_ref[...], preferred_element_type=jnp.float32)
              + bnb_ref[...])
        att = jnp.sum(x * w2, axis=1, keepdims=True)
        e = jnp.exp(att - jnp.max(att))
        m_ref[:, b * F:(b + 1) * F] = e * nb
        es.append(e)
    es.append(jnp.zeros((N, F - B), dtype=jnp.float32))
    m_ref[:, B * F:] = jnp.concatenate(es, axis=1)


def _main_body(x_ref, a_ref, m_ref, wself_ref, bself_ref, gamma_ref, beta_ref,
               out_ref):
    B = x_ref.shape[0]
    F = x_ref.shape[2]
    mm = jnp.dot(a_ref[...], m_ref[...], preferred_element_type=jnp.float32)
    for b in range(B):
        self_blk = (jnp.dot(x_ref[b], wself_ref[...],
                            preferred_element_type=jnp.float32)
                    + bself_ref[...])
        num = mm[:, b * F:(b + 1) * F]
        den = mm[:, B * F + b:B * F + b + 1]
        rec = jnp.where(den > 0, 1.0 / den, 0.0)
        comb = self_blk + num * rec
        mean = jnp.mean(comb, axis=1, keepdims=True)
        cent = comb - mean
        var = jnp.mean(cent * cent, axis=1, keepdims=True)
        rstd = jax.lax.rsqrt(var + 1e-5)
        out_ref[b] = jnp.maximum(
            (cent * rstd) * gamma_ref[...] + beta_ref[...], 0.0)


def kernel(node_features, adjacency_matrix, W_self, b_self, W_nb, b_nb,
           W_att, b_att, ln_gamma, ln_beta):
    B, N, F = node_features.shape
    watt2 = W_att.reshape(2, F)
    bself = b_self.reshape(1, F)
    bnb = b_nb.reshape(1, F)
    gamma = ln_gamma.reshape(1, F)
    beta = ln_beta.reshape(1, F)
    MCOLS = (B + 1) * F

    m = pl.pallas_call(
        _prep_body,
        in_specs=[
            pl.BlockSpec((B, N, F), lambda: (0, 0, 0)),
            pl.BlockSpec((F, F), lambda: (0, 0)),
            pl.BlockSpec((1, F), lambda: (0, 0)),
            pl.BlockSpec((2, F), lambda: (0, 0)),
        ],
        out_specs=pl.BlockSpec((N, MCOLS), lambda: (0, 0)),
        out_shape=jax.ShapeDtypeStruct((N, MCOLS), jnp.float32),
    )(node_features, W_nb, bnb, watt2)

    grid = (N // _BLK,)
    out = pl.pallas_call(
        _main_body,
        grid=grid,
        in_specs=[
            pl.BlockSpec((B, _BLK, F), lambda i: (0, i, 0)),
            pl.BlockSpec((_BLK, N), lambda i: (i, 0)),
            pl.BlockSpec((N, MCOLS), lambda i: (0, 0)),
            pl.BlockSpec((F, F), lambda i: (0, 0)),
            pl.BlockSpec((1, F), lambda i: (0, 0)),
            pl.BlockSpec((1, F), lambda i: (0, 0)),
            pl.BlockSpec((1, F), lambda i: (0, 0)),
        ],
        out_specs=pl.BlockSpec((B, _BLK, F), lambda i: (0, i, 0)),
        out_shape=jax.ShapeDtypeStruct((B, N, F), jnp.float32),
        compiler_params=pltpu.CompilerParams(
            dimension_semantics=("parallel",),
        ),
    )(node_features, adjacency_matrix, m, W_self, bself, gamma, beta)
    return out


# speedup vs baseline: 1.1479x; 1.1456x over previous
import jax
import jax.numpy as jnp
from jax.experimental import pallas as pl
from jax.experimental.pallas import tpu as pltpu

_BLK = 256


def _fused_body(x_ref, a_ref, wself_ref, bself_ref, wnb_ref, bnb_ref, watt_ref,
                gamma_ref, beta_ref, out_ref, m_scr, waug_scr, bm_scr):
    i = pl.program_id(0)
    B, N, F = x_ref.shape

    @pl.when(i == 0)
    def _prep():
        w2 = watt_ref[1:2, :]
        es = []
        ems = []
        for b in range(B):
            x = x_ref[b]
            nb = (jnp.dot(x, wnb_ref[...], preferred_element_type=jnp.float32)
                  + bnb_ref[...])
            att = jnp.sum(x * w2, axis=1, keepdims=True)
            e = jnp.exp(att - jnp.max(att))
            m_scr[:, b * F:(b + 1) * F] = e * nb
            es.append(e)
            ems.append(e * jnp.mean(nb, axis=1, keepdims=True))
        tail = es + ems
        tail.append(jnp.zeros((N, F - 2 * B), dtype=jnp.float32))
        m_scr[:, B * F:] = jnp.concatenate(tail, axis=1)
        wcols = [wself_ref[...],
                 jnp.mean(wself_ref[...], axis=1, keepdims=True),
                 jnp.zeros((F, F - 1), dtype=jnp.float32)]
        waug_scr[...] = jnp.concatenate(wcols, axis=1)
        bm_scr[0] = jnp.mean(bself_ref[...])

    mm = jnp.dot(a_ref[...], m_scr[...], preferred_element_type=jnp.float32)
    bm = bm_scr[0]
    for b in range(B):
        x_blk = x_ref[b, pl.ds(i * _BLK, _BLK), :]
        saug = jnp.dot(x_blk, waug_scr[...], preferred_element_type=jnp.float32)
        self_blk = saug[:, :F] + bself_ref[...]
        num = mm[:, b * F:(b + 1) * F]
        den = mm[:, B * F + b:B * F + b + 1]
        mnum = mm[:, B * F + B + b:B * F + B + b + 1]
        rec = jnp.where(den > 0, 1.0 / den, 0.0)
        comb = self_blk + num * rec
        mean = saug[:, F:F + 1] + bm + mnum * rec
        cent = comb - mean
        var = jnp.mean(cent * cent, axis=1, keepdims=True)
        rstd = jax.lax.rsqrt(var + 1e-5)
        out_ref[b] = jnp.maximum(
            (cent * rstd) * gamma_ref[...] + beta_ref[...], 0.0)


def kernel(node_features, adjacency_matrix, W_self, b_self, W_nb, b_nb,
           W_att, b_att, ln_gamma, ln_beta):
    B, N, F = node_features.shape
    watt2 = W_att.reshape(2, F)
    bself = b_self.reshape(1, F)
    bnb = b_nb.reshape(1, F)
    gamma = ln_gamma.reshape(1, F)
    beta = ln_beta.reshape(1, F)

    grid = (N // _BLK,)
    out = pl.pallas_call(
        _fused_body,
        grid=grid,
        in_specs=[
            pl.BlockSpec((B, N, F), lambda i: (0, 0, 0)),
            pl.BlockSpec((_BLK, N), lambda i: (i, 0)),
            pl.BlockSpec((F, F), lambda i: (0, 0)),
            pl.BlockSpec((1, F), lambda i: (0, 0)),
            pl.BlockSpec((F, F), lambda i: (0, 0)),
            pl.BlockSpec((1, F), lambda i: (0, 0)),
            pl.BlockSpec((2, F), lambda i: (0, 0)),
            pl.BlockSpec((1, F), lambda i: (0, 0)),
            pl.BlockSpec((1, F), lambda i: (0, 0)),
        ],
        out_specs=pl.BlockSpec((B, _BLK, F), lambda i: (0, i, 0)),
        out_shape=jax.ShapeDtypeStruct((B, N, F), jnp.float32),
        scratch_shapes=[
            pltpu.VMEM((N, (B + 1) * F), jnp.float32),
            pltpu.VMEM((F, 2 * F), jnp.float32),
            pltpu.SMEM((1,), jnp.float32),
        ],
        compiler_params=pltpu.CompilerParams(
            dimension_semantics=("arbitrary",),
        ),
    )(node_features, adjacency_matrix, W_self, bself, W_nb, bnb, watt2,
      gamma, beta)
    return out


# speedup vs baseline: 1.2026x; 1.0477x over previous
import jax
import jax.numpy as jnp
from jax.experimental import pallas as pl
from jax.experimental.pallas import tpu as pltpu

_BLK = 256


def _fused_body(x_ref, a_ref, wself_ref, bself_ref, wnb_ref, bnb_ref, watt_ref,
                gamma_ref, beta_ref, out_ref, m_scr, e_scr):
    i = pl.program_id(0)
    B, N, F = x_ref.shape

    @pl.when(i == 0)
    def _prep():
        w2 = watt_ref[1:2, :]
        es = []
        for b in range(B):
            x = x_ref[b]
            nb = (jnp.dot(x, wnb_ref[...], preferred_element_type=jnp.float32)
                  + bnb_ref[...])
            att = jnp.sum(x * w2, axis=1, keepdims=True)
            e = jnp.exp(att - jnp.max(att))
            m_scr[b] = e * nb
            es.append(e)
        es.append(jnp.zeros((N, F - B), dtype=jnp.float32))
        e_scr[...] = jnp.concatenate(es, axis=1)

    a_blk = a_ref[...]
    den_all = jnp.dot(a_blk, e_scr[...], preferred_element_type=jnp.float32)
    for b in range(B):
        x_blk = x_ref[b, pl.ds(i * _BLK, _BLK), :]
        self_blk = (jnp.dot(x_blk, wself_ref[...],
                            preferred_element_type=jnp.float32)
                    + bself_ref[...])
        num = jnp.dot(a_blk, m_scr[b], preferred_element_type=jnp.float32)
        den = den_all[:, b:b + 1]
        rec = jnp.where(den > 0, 1.0 / den, 0.0)
        comb = self_blk + num * rec
        mean = jnp.mean(comb, axis=1, keepdims=True)
        cent = comb - mean
        var = jnp.mean(cent * cent, axis=1, keepdims=True)
        rstd = jax.lax.rsqrt(var + 1e-5)
        out_ref[b] = jnp.maximum(
            (cent * rstd) * gamma_ref[...] + beta_ref[...], 0.0)


def kernel(node_features, adjacency_matrix, W_self, b_self, W_nb, b_nb,
           W_att, b_att, ln_gamma, ln_beta):
    B, N, F = node_features.shape
    watt2 = W_att.reshape(2, F)
    bself = b_self.reshape(1, F)
    bnb = b_nb.reshape(1, F)
    gamma = ln_gamma.reshape(1, F)
    beta = ln_beta.reshape(1, F)

    grid = (N // _BLK,)
    out = pl.pallas_call(
        _fused_body,
        grid=grid,
        in_specs=[
            pl.BlockSpec((B, N, F), lambda i: (0, 0, 0)),
            pl.BlockSpec((_BLK, N), lambda i: (i, 0)),
            pl.BlockSpec((F, F), lambda i: (0, 0)),
            pl.BlockSpec((1, F), lambda i: (0, 0)),
            pl.BlockSpec((F, F), lambda i: (0, 0)),
            pl.BlockSpec((1, F), lambda i: (0, 0)),
            pl.BlockSpec((2, F), lambda i: (0, 0)),
            pl.BlockSpec((1, F), lambda i: (0, 0)),
            pl.BlockSpec((1, F), lambda i: (0, 0)),
        ],
        out_specs=pl.BlockSpec((B, _BLK, F), lambda i: (0, i, 0)),
        out_shape=jax.ShapeDtypeStruct((B, N, F), jnp.float32),
        scratch_shapes=[
            pltpu.VMEM((B, N, F), jnp.float32),
            pltpu.VMEM((N, F), jnp.float32),
        ],
        compiler_params=pltpu.CompilerParams(
            dimension_semantics=("arbitrary",),
        ),
    )(node_features, adjacency_matrix, W_self, bself, W_nb, bnb, watt2,
      gamma, beta)
    return out


# speedup vs baseline: 1.2559x; 1.0443x over previous
import jax
import jax.numpy as jnp
from jax.experimental import pallas as pl
from jax.experimental.pallas import tpu as pltpu

_BLK = 256


def _fused_body(x_ref, a_ref, wself_ref, bself_ref, wnb_ref, bnb_ref, watt_ref,
                gamma_ref, beta_ref, out_ref, m_scr):
    i = pl.program_id(0)
    B, N, F = x_ref.shape

    @pl.when(i == 0)
    def _prep():
        w2 = watt_ref[1:2, :]
        es = []
        for b in range(B):
            x = x_ref[b]
            nb = (jnp.dot(x, wnb_ref[...], preferred_element_type=jnp.float32)
                  + bnb_ref[...])
            att = jnp.sum(x * w2, axis=1, keepdims=True)
            e = jnp.exp(att - jnp.max(att))
            m_scr[:, b * F:(b + 1) * F] = (e * nb).astype(m_scr.dtype)
            es.append(e)
        es.append(jnp.zeros((N, F - B), dtype=jnp.float32))
        m_scr[:, B * F:] = jnp.concatenate(es, axis=1).astype(m_scr.dtype)

    mm = jnp.dot(a_ref[...].astype(m_scr.dtype), m_scr[...],
                 preferred_element_type=jnp.float32)
    for b in range(B):
        x_blk = x_ref[b, pl.ds(i * _BLK, _BLK), :]
        self_blk = (jnp.dot(x_blk, wself_ref[...],
                            preferred_element_type=jnp.float32)
                    + bself_ref[...])
        num = mm[:, b * F:(b + 1) * F]
        den = mm[:, B * F + b:B * F + b + 1]
        rec = jnp.where(den > 0, 1.0 / den, 0.0)
        comb = self_blk + num * rec
        mean = jnp.mean(comb, axis=1, keepdims=True)
        cent = comb - mean
        var = jnp.mean(cent * cent, axis=1, keepdims=True)
        rstd = jax.lax.rsqrt(var + 1e-5)
        out_ref[b] = jnp.maximum(
            (cent * rstd) * gamma_ref[...] + beta_ref[...], 0.0)


def kernel(node_features, adjacency_matrix, W_self, b_self, W_nb, b_nb,
           W_att, b_att, ln_gamma, ln_beta):
    B, N, F = node_features.shape
    watt2 = W_att.reshape(2, F)
    bself = b_self.reshape(1, F)
    bnb = b_nb.reshape(1, F)
    gamma = ln_gamma.reshape(1, F)
    beta = ln_beta.reshape(1, F)

    grid = (N // _BLK,)
    out = pl.pallas_call(
        _fused_body,
        grid=grid,
        in_specs=[
            pl.BlockSpec((B, N, F), lambda i: (0, 0, 0)),
            pl.BlockSpec((_BLK, N), lambda i: (i, 0)),
            pl.BlockSpec((F, F), lambda i: (0, 0)),
            pl.BlockSpec((1, F), lambda i: (0, 0)),
            pl.BlockSpec((F, F), lambda i: (0, 0)),
            pl.BlockSpec((1, F), lambda i: (0, 0)),
            pl.BlockSpec((2, F), lambda i: (0, 0)),
            pl.BlockSpec((1, F), lambda i: (0, 0)),
            pl.BlockSpec((1, F), lambda i: (0, 0)),
        ],
        out_specs=pl.BlockSpec((B, _BLK, F), lambda i: (0, i, 0)),
        out_shape=jax.ShapeDtypeStruct((B, N, F), jnp.float32),
        scratch_shapes=[
            pltpu.VMEM((N, (B + 1) * F), jnp.bfloat16),
        ],
        compiler_params=pltpu.CompilerParams(
            dimension_semantics=("arbitrary",),
        ),
    )(node_features, adjacency_matrix, W_self, bself, W_nb, bnb, watt2,
      gamma, beta)
    return out
